# speedup vs baseline: 3.3457x; 3.3457x over previous
import jax
import jax.numpy as jnp
from jax import lax
from jax.experimental import pallas as pl
from jax.experimental.pallas import tpu as pltpu
from jax.experimental.pallas import tpu_sc as plsc

_D = 128
_B_TOTAL = 4096 * 50
_NW = 32
_B_PER_W = _B_TOTAL // _NW
_C = 128
_NCHUNK = _B_PER_W // _C


def _gather_body(idx_hbm, table_hbm, out_hbm, idx_v, bufs, gsem):
    cid = lax.axis_index("c")
    sid = lax.axis_index("s")
    wid = sid * 2 + cid
    base = wid * _B_PER_W

    pltpu.sync_copy(idx_hbm.at[pl.ds(base, _B_PER_W)], idx_v)

    def gather(j, b):
        src = table_hbm.at[idx_v.at[pl.ds(j * _C, _C)]]
        return pltpu.make_async_copy(src, bufs.at[b], gsem.at[b])

    gather(0, 0).start()

    def body(j, carry):
        b = lax.rem(j, 2)
        jn = j + 1

        @pl.when(jn < _NCHUNK)
        def _():
            gather(jn, 1 - b).start()

        gather(j, b).wait()
        pltpu.sync_copy(bufs.at[b], out_hbm.at[pl.ds(base + j * _C, _C)])
        return carry

    lax.fori_loop(0, _NCHUNK, body, None)


_mesh = plsc.VectorSubcoreMesh(core_axis_name="c", subcore_axis_name="s")


@jax.jit
def _embed_lookup(idx_flat, table):
    return pl.kernel(
        _gather_body,
        out_type=jax.ShapeDtypeStruct((_B_TOTAL, _D), jnp.float32),
        mesh=_mesh,
        scratch_types=[
            pltpu.VMEM((_B_PER_W,), jnp.int32),
            pltpu.VMEM((2, _C, _D), jnp.float32),
            pltpu.SemaphoreType.DMA((2,)),
        ],
    )(idx_flat, table)


def kernel(inputs, embedding):
    idx_flat = inputs.reshape(-1).astype(jnp.int32)
    out = _embed_lookup(idx_flat, embedding)
    return out.reshape(inputs.shape + (_D,))
